# baseline (speedup 1.0000x reference)
import functools

import jax
import jax.numpy as jnp
from jax import lax
from jax.experimental import pallas as pl
from jax.experimental.pallas import tpu as pltpu
from jax.experimental.pallas import tpu_sc as plsc

E = 4
D = 2048
R = 256
T = 4096
MAXB = T // R + E
CAP = MAXB * R

TROWS = 32
TLANES = 128

NW = 32
TOK_PER_W = T // NW
CH = 16
NCH = TOK_PER_W // CH


def _routing_body(tid_ref, p_ref, gid_ref):
    tid = tid_ref[...]
    rk = lax.broadcasted_iota(jnp.int32, (TLANES, TLANES), 0)
    ck = lax.broadcasted_iota(jnp.int32, (TLANES, TLANES), 1)
    upper_incl = (rk <= ck).astype(jnp.float32)
    rr = lax.broadcasted_iota(jnp.int32, (TROWS, TROWS), 0)
    cr = lax.broadcasted_iota(jnp.int32, (TROWS, TROWS), 1)
    lower_strict = (cr < rr).astype(jnp.float32)

    ranks = []
    counts = []
    for g in range(E):
        m = (tid == g).astype(jnp.float32)
        lane_cum = jnp.dot(m, upper_incl, preferred_element_type=jnp.float32,
                           precision=lax.Precision.HIGHEST)
        row_tot = jnp.sum(m, axis=1, keepdims=True)
        row_excl = jnp.dot(lower_strict, row_tot,
                           preferred_element_type=jnp.float32,
                           precision=lax.Precision.HIGHEST)
        ranks.append(row_excl + lane_cum - 1.0)
        counts.append(jnp.sum(m))

    p = jnp.zeros((TROWS, TLANES), jnp.float32)
    base = jnp.float32(0.0)
    nblk_cum = []
    acc = jnp.float32(0.0)
    for g in range(E):
        p = jnp.where(tid == g, base + ranks[g], p)
        nblk = jnp.ceil(counts[g] / R)
        base = base + nblk * R
        acc = acc + nblk
        nblk_cum.append(acc)
    p_ref[...] = p.astype(jnp.int32)

    ii = lax.broadcasted_iota(jnp.int32, (1, TLANES), 1).astype(jnp.float32)
    gid = jnp.zeros((1, TLANES), jnp.float32)
    for g in range(E):
        gid = gid + (ii >= nblk_cum[g]).astype(jnp.float32)
    gid = jnp.minimum(gid, float(E - 1))
    gid = jnp.where(ii == float(MAXB), nblk_cum[E - 1], gid)
    gid_ref[...] = gid.astype(jnp.int32)


_routing = pl.pallas_call(
    _routing_body,
    out_shape=(
        jax.ShapeDtypeStruct((TROWS, TLANES), jnp.int32),
        jax.ShapeDtypeStruct((1, TLANES), jnp.int32),
    ),
)


def _mm_body(gid_ref, x_ref, w_ref, b_ref, y_ref):
    @pl.when(pl.program_id(0) < gid_ref[MAXB])
    def _():
        x16 = x_ref[...].astype(jnp.bfloat16)
        w16 = w_ref[0].astype(jnp.bfloat16)
        y_ref[...] = (
            jnp.dot(x16, w16, preferred_element_type=jnp.float32)
            + b_ref[0]
        )


_grouped_mm = pl.pallas_call(
    _mm_body,
    grid_spec=pltpu.PrefetchScalarGridSpec(
        num_scalar_prefetch=1,
        grid=(MAXB,),
        in_specs=[
            pl.BlockSpec((R, D), lambda i, gid: (i, 0)),
            pl.BlockSpec((1, D, D), lambda i, gid: (gid[i], 0, 0)),
            pl.BlockSpec((1, 1, D), lambda i, gid: (gid[i], 0, 0)),
        ],
        out_specs=pl.BlockSpec((R, D), lambda i, gid: (i, 0)),
    ),
    out_shape=jax.ShapeDtypeStruct((CAP, D), jnp.float32),
)

@functools.cache
def _sc_kernels():
    mesh = plsc.VectorSubcoreMesh(core_axis_name="c", subcore_axis_name="s")
    scratch = [
        pltpu.VMEM((NCH, CH), jnp.int32),
        pltpu.VMEM((CH, D), jnp.float32),
        pltpu.VMEM((CH, D), jnp.float32),
        pltpu.SemaphoreType.DMA,
        pltpu.SemaphoreType.DMA,
        pltpu.SemaphoreType.DMA,
    ]

    @functools.partial(
        pl.kernel,
        out_type=jax.ShapeDtypeStruct((CAP, D), jnp.float32),
        mesh=mesh,
        scratch_types=scratch,
    )
    def sc_scatter(h_hbm, p_hbm, xs_hbm, idx_v, buf0, buf1, lsem0, lsem1, ssem):
        wid = lax.axis_index("s") * 2 + lax.axis_index("c")
        base = wid * TOK_PER_W
        bufs = (buf0, buf1)
        lsems = (lsem0, lsem1)
        pltpu.sync_copy(p_hbm.at[pl.ds(wid * NCH, NCH)], idx_v)
        pltpu.async_copy(h_hbm.at[pl.ds(base, CH)], buf0, lsem0)
        pltpu.async_copy(h_hbm.at[pl.ds(base + CH, CH)], buf1, lsem1)
        for c in range(NCH):
            b = c % 2
            pltpu.make_async_copy(
                h_hbm.at[pl.ds(base, CH)], bufs[b], lsems[b]
            ).wait()
            pltpu.async_copy(bufs[b], xs_hbm.at[idx_v.at[c]], ssem).wait()
            if c + 2 < NCH:
                pltpu.async_copy(
                    h_hbm.at[pl.ds(base + (c + 2) * CH, CH)], bufs[b], lsems[b]
                )

    @functools.partial(
        pl.kernel,
        out_type=jax.ShapeDtypeStruct((T, D), jnp.float32),
        mesh=mesh,
        scratch_types=scratch,
    )
    def sc_gather(y_hbm, p_hbm, out_hbm, idx_v, buf0, buf1, gsem0, gsem1, wsem):
        wid = lax.axis_index("s") * 2 + lax.axis_index("c")
        base = wid * TOK_PER_W
        bufs = (buf0, buf1)
        gsems = (gsem0, gsem1)
        pltpu.sync_copy(p_hbm.at[pl.ds(wid * NCH, NCH)], idx_v)
        pltpu.async_copy(y_hbm.at[idx_v.at[0]], buf0, gsem0)
        pltpu.async_copy(y_hbm.at[idx_v.at[1]], buf1, gsem1)
        for c in range(NCH):
            b = c % 2
            pltpu.make_async_copy(
                y_hbm.at[idx_v.at[c]], bufs[b], gsems[b]
            ).wait()
            pltpu.async_copy(
                bufs[b], out_hbm.at[pl.ds(base + c * CH, CH)], wsem
            ).wait()
            if c + 2 < NCH:
                pltpu.async_copy(y_hbm.at[idx_v.at[c + 2]], bufs[b], gsems[b])

    return sc_scatter, sc_gather


@jax.jit
def kernel(hidden_states, type_ids, W, b):
    B, S, _ = hidden_states.shape
    h2d = hidden_states.reshape(T, D)
    tid = type_ids.reshape(TROWS, TLANES).astype(jnp.int32)
    p2d, gid_row = _routing(tid)
    p_chunks = p2d.reshape(T // CH, CH)
    block_gid = gid_row[0, : MAXB + 1]
    sc_scatter, sc_gather = _sc_kernels()
    x_sorted = sc_scatter(h2d, p_chunks)
    y_sorted = _grouped_mm(block_gid, x_sorted, W, b.reshape(E, 1, D))
    out = sc_gather(y_sorted, p_chunks)
    return out.reshape(B, S, D)


# speedup vs baseline: 1.0234x; 1.0234x over previous
import functools

import jax
import jax.numpy as jnp
from jax import lax
from jax.experimental import pallas as pl
from jax.experimental.pallas import tpu as pltpu
from jax.experimental.pallas import tpu_sc as plsc

E = 4
D = 2048
R = 256
T = 4096
MAXB = T // R + E - 1
CAP = MAXB * R

TROWS = 32
TLANES = 128

NW = 32
TOK_PER_W = T // NW
CH = 8
NCH = TOK_PER_W // CH
NBUF = 6
DEPTH = 3


def _routing_body(tid_ref, p_ref, gid_ref):
    tid = tid_ref[...]
    rk = lax.broadcasted_iota(jnp.int32, (TLANES, TLANES), 0)
    ck = lax.broadcasted_iota(jnp.int32, (TLANES, TLANES), 1)
    upper_incl = (rk <= ck).astype(jnp.float32)
    rr = lax.broadcasted_iota(jnp.int32, (TROWS, TROWS), 0)
    cr = lax.broadcasted_iota(jnp.int32, (TROWS, TROWS), 1)
    lower_strict = (cr < rr).astype(jnp.float32)

    ranks = []
    counts = []
    for g in range(E):
        m = (tid == g).astype(jnp.float32)
        lane_cum = jnp.dot(m, upper_incl, preferred_element_type=jnp.float32,
                           precision=lax.Precision.HIGHEST)
        row_tot = jnp.sum(m, axis=1, keepdims=True)
        row_excl = jnp.dot(lower_strict, row_tot,
                           preferred_element_type=jnp.float32,
                           precision=lax.Precision.HIGHEST)
        ranks.append(row_excl + lane_cum - 1.0)
        counts.append(jnp.sum(m))

    p = jnp.zeros((TROWS, TLANES), jnp.float32)
    base = jnp.float32(0.0)
    nblk_cum = []
    acc = jnp.float32(0.0)
    for g in range(E):
        p = jnp.where(tid == g, base + ranks[g], p)
        nblk = jnp.ceil(counts[g] / R)
        base = base + nblk * R
        acc = acc + nblk
        nblk_cum.append(acc)
    p_ref[...] = p.astype(jnp.int32)

    ii = lax.broadcasted_iota(jnp.int32, (1, TLANES), 1).astype(jnp.float32)
    gid = jnp.zeros((1, TLANES), jnp.float32)
    for g in range(E):
        gid = gid + (ii >= nblk_cum[g]).astype(jnp.float32)
    gid = jnp.minimum(gid, float(E - 1))
    gid = jnp.where(ii == float(MAXB), nblk_cum[E - 1], gid)
    gid_ref[...] = gid.astype(jnp.int32)


_routing = pl.pallas_call(
    _routing_body,
    out_shape=(
        jax.ShapeDtypeStruct((TROWS, TLANES), jnp.int32),
        jax.ShapeDtypeStruct((1, TLANES), jnp.int32),
    ),
)


def _mm_body(gid_ref, x_ref, w_ref, b_ref, y_ref):
    @pl.when(pl.program_id(0) < gid_ref[MAXB])
    def _():
        y_ref[...] = (
            jnp.dot(x_ref[...], w_ref[0], preferred_element_type=jnp.float32)
            + b_ref[0]
        )


_grouped_mm = pl.pallas_call(
    _mm_body,
    grid_spec=pltpu.PrefetchScalarGridSpec(
        num_scalar_prefetch=1,
        grid=(MAXB,),
        in_specs=[
            pl.BlockSpec((R, D), lambda i, gid: (i, 0)),
            pl.BlockSpec((1, D, D), lambda i, gid: (gid[i], 0, 0)),
            pl.BlockSpec((1, 1, D), lambda i, gid: (gid[i], 0, 0)),
        ],
        out_specs=pl.BlockSpec((R, D), lambda i, gid: (i, 0)),
    ),
    out_shape=jax.ShapeDtypeStruct((CAP, D), jnp.float32),
)

@functools.cache
def _sc_kernels():
    mesh = plsc.VectorSubcoreMesh(core_axis_name="c", subcore_axis_name="s")
    scratch = (
        [pltpu.VMEM((NCH, CH), jnp.int32)]
        + [pltpu.VMEM((CH, D), jnp.float32) for _ in range(NBUF)]
        + [pltpu.SemaphoreType.DMA for _ in range(2 * NBUF)]
    )

    @functools.partial(
        pl.kernel,
        out_type=jax.ShapeDtypeStruct((CAP, D), jnp.float32),
        mesh=mesh,
        scratch_types=scratch,
    )
    def sc_scatter(h_hbm, p_hbm, xs_hbm, idx_v, *bufs_sems):
        bufs = bufs_sems[:NBUF]
        lsems = bufs_sems[NBUF : NBUF + NBUF]
        ssems = bufs_sems[NBUF + NBUF :]
        wid = lax.axis_index("s") * 2 + lax.axis_index("c")
        base = wid * TOK_PER_W
        pltpu.sync_copy(p_hbm.at[pl.ds(wid * NCH, NCH)], idx_v)
        for j in range(DEPTH):
            pltpu.async_copy(
                h_hbm.at[pl.ds(base + j * CH, CH)], bufs[j], lsems[j]
            )
        for c in range(NCH):
            b = c % NBUF
            pltpu.make_async_copy(
                h_hbm.at[pl.ds(base, CH)], bufs[b], lsems[b]
            ).wait()
            pltpu.async_copy(bufs[b], xs_hbm.at[idx_v.at[c]], ssems[b])
            nxt = c + DEPTH
            if nxt < NCH:
                bb = nxt % NBUF
                if nxt >= NBUF:
                    pltpu.make_async_copy(
                        bufs[bb], xs_hbm.at[idx_v.at[c]], ssems[bb]
                    ).wait()
                pltpu.async_copy(
                    h_hbm.at[pl.ds(base + nxt * CH, CH)], bufs[bb], lsems[bb]
                )
        for c in range(NCH - NBUF, NCH):
            b = c % NBUF
            pltpu.make_async_copy(
                bufs[b], xs_hbm.at[idx_v.at[c]], ssems[b]
            ).wait()

    @functools.partial(
        pl.kernel,
        out_type=jax.ShapeDtypeStruct((T, D), jnp.float32),
        mesh=mesh,
        scratch_types=scratch,
    )
    def sc_gather(y_hbm, p_hbm, out_hbm, idx_v, *bufs_sems):
        bufs = bufs_sems[:NBUF]
        gsems = bufs_sems[NBUF : NBUF + NBUF]
        wsems = bufs_sems[NBUF + NBUF :]
        wid = lax.axis_index("s") * 2 + lax.axis_index("c")
        base = wid * TOK_PER_W
        pltpu.sync_copy(p_hbm.at[pl.ds(wid * NCH, NCH)], idx_v)
        for j in range(DEPTH):
            pltpu.async_copy(y_hbm.at[idx_v.at[j]], bufs[j], gsems[j])
        for c in range(NCH):
            b = c % NBUF
            pltpu.make_async_copy(
                y_hbm.at[idx_v.at[c]], bufs[b], gsems[b]
            ).wait()
            pltpu.async_copy(
                bufs[b], out_hbm.at[pl.ds(base + c * CH, CH)], wsems[b]
            )
            nxt = c + DEPTH
            if nxt < NCH:
                bb = nxt % NBUF
                if nxt >= NBUF:
                    pltpu.make_async_copy(
                        bufs[bb], out_hbm.at[pl.ds(base, CH)], wsems[bb]
                    ).wait()
                pltpu.async_copy(y_hbm.at[idx_v.at[nxt]], bufs[bb], gsems[bb])
        for c in range(NCH - NBUF, NCH):
            b = c % NBUF
            pltpu.make_async_copy(
                bufs[b], out_hbm.at[pl.ds(base, CH)], wsems[b]
            ).wait()

    return sc_scatter, sc_gather


@jax.jit
def kernel(hidden_states, type_ids, W, b):
    B, S, _ = hidden_states.shape
    h2d = hidden_states.reshape(T, D)
    tid = type_ids.reshape(TROWS, TLANES).astype(jnp.int32)
    p2d, gid_row = _routing(tid)
    p_chunks = p2d.reshape(T // CH, CH)
    block_gid = gid_row[0, : MAXB + 1]
    sc_scatter, sc_gather = _sc_kernels()
    x_sorted = sc_scatter(h2d, p_chunks)
    y_sorted = _grouped_mm(block_gid, x_sorted, W, b.reshape(E, 1, D))
    out = sc_gather(y_sorted, p_chunks)
    return out.reshape(B, S, D)


# speedup vs baseline: 1.0559x; 1.0317x over previous
import functools

import jax
import jax.numpy as jnp
from jax import lax
from jax.experimental import pallas as pl
from jax.experimental.pallas import tpu as pltpu
from jax.experimental.pallas import tpu_sc as plsc

E = 4
D = 2048
R = 256
T = 4096
MAXB = T // R + E - 1
CAP = MAXB * R

TROWS = 32
TLANES = 128

NW = 32
TOK_PER_W = T // NW
CH = 8
NCH = TOK_PER_W // CH
NBUF = 6
DEPTH = 3


def _routing_body(tid_ref, p_ref, gid_ref):
    tid = tid_ref[...]
    rk = lax.broadcasted_iota(jnp.int32, (TLANES, TLANES), 0)
    ck = lax.broadcasted_iota(jnp.int32, (TLANES, TLANES), 1)
    upper_incl = (rk <= ck).astype(jnp.float32)
    rr = lax.broadcasted_iota(jnp.int32, (TROWS, TROWS), 0)
    cr = lax.broadcasted_iota(jnp.int32, (TROWS, TROWS), 1)
    lower_strict = (cr < rr).astype(jnp.float32)

    ranks = []
    counts = []
    for g in range(E):
        m = (tid == g).astype(jnp.float32)
        lane_cum = jnp.dot(m, upper_incl, preferred_element_type=jnp.float32,
                           precision=lax.Precision.HIGHEST)
        row_tot = jnp.sum(m, axis=1, keepdims=True)
        row_excl = jnp.dot(lower_strict, row_tot,
                           preferred_element_type=jnp.float32,
                           precision=lax.Precision.HIGHEST)
        ranks.append(row_excl + lane_cum - 1.0)
        counts.append(jnp.sum(m))

    p = jnp.zeros((TROWS, TLANES), jnp.float32)
    base = jnp.float32(0.0)
    nblk_cum = []
    acc = jnp.float32(0.0)
    for g in range(E):
        p = jnp.where(tid == g, base + ranks[g], p)
        nblk = jnp.ceil(counts[g] / R)
        base = base + nblk * R
        acc = acc + nblk
        nblk_cum.append(acc)
    p_ref[...] = p.astype(jnp.int32)

    ii = lax.broadcasted_iota(jnp.int32, (1, TLANES), 1).astype(jnp.float32)
    gid = jnp.zeros((1, TLANES), jnp.float32)
    for g in range(E):
        gid = gid + (ii >= nblk_cum[g]).astype(jnp.float32)
    gid = jnp.minimum(gid, float(E - 1))
    nb = nblk_cum[E - 1]

    prev = jnp.concatenate([gid[:, :1], gid[:, :-1]], axis=1)
    valid = ii < nb
    chg = jnp.where((gid != prev) | (ii == 0.0), 1.0, 0.0) * valid
    cs = jnp.dot(chg, upper_incl, preferred_element_type=jnp.float32,
                 precision=lax.Precision.HIGHEST)
    parity = cs - 1.0 - 2.0 * jnp.floor((cs - 1.0) / 2.0)
    np3 = jnp.float32(3.0)
    np2 = jnp.where(counts[3] > 0, 3.0, 2.0)
    np1 = jnp.where(counts[2] > 0, 2.0, jnp.where(counts[3] > 0, 3.0, 1.0))
    np0 = jnp.where(
        counts[1] > 0, 1.0,
        jnp.where(counts[2] > 0, 2.0, jnp.where(counts[3] > 0, 3.0, 0.0)),
    )
    nps = [np0, np1, np2, np3]
    nxt = jnp.zeros((1, TLANES), jnp.float32)
    for g in range(E):
        nxt = jnp.where(gid == g, nps[g], nxt)

    row0 = jnp.where(ii == float(MAXB), nb, gid)
    aux = jnp.concatenate([row0, parity, chg, nxt], axis=0)
    gid_ref[...] = aux.astype(jnp.int32)


_routing = pl.pallas_call(
    _routing_body,
    out_shape=(
        jax.ShapeDtypeStruct((TROWS, TLANES), jnp.int32),
        jax.ShapeDtypeStruct((4, TLANES), jnp.int32),
    ),
)


def _mm_body(aux_ref, x_ref, w_hbm, b_ref, y_ref, w_vmem, sem0, sem1):
    i = pl.program_id(0)
    nb = aux_ref[0, MAXB]
    gid = aux_ref[0, i]
    par = aux_ref[1, i]
    start = aux_ref[2, i]
    nxt = aux_ref[3, i]
    real = i < nb
    sems = (sem0, sem1)

    def copy_w(g_idx, slot):
        pltpu.make_async_copy(w_hbm.at[g_idx], w_vmem.at[slot], sems[slot]).start()

    @pl.when(real & (start == 1) & (i == 0))
    def _():
        copy_w(gid, 0)

    for s in (0, 1):
        @pl.when(real & (start == 1) & (par == s))
        def _(s=s):
            pltpu.make_async_copy(w_hbm.at[0], w_vmem.at[s], sems[s]).wait()

            @pl.when(nxt != gid)
            def _():
                copy_w(nxt, 1 - s)

    for s in (0, 1):
        @pl.when(real & (par == s))
        def _(s=s):
            y_ref[...] = (
                jnp.dot(x_ref[...], w_vmem[s],
                        preferred_element_type=jnp.float32)
                + b_ref[0]
            )


_grouped_mm = pl.pallas_call(
    _mm_body,
    grid_spec=pltpu.PrefetchScalarGridSpec(
        num_scalar_prefetch=1,
        grid=(MAXB,),
        in_specs=[
            pl.BlockSpec((R, D), lambda i, aux: (i, 0)),
            pl.BlockSpec(memory_space=pl.ANY),
            pl.BlockSpec((1, 1, D), lambda i, aux: (aux[0, i], 0, 0)),
        ],
        out_specs=pl.BlockSpec((R, D), lambda i, aux: (i, 0)),
        scratch_shapes=[
            pltpu.VMEM((2, D, D), jnp.float32),
            pltpu.SemaphoreType.DMA,
            pltpu.SemaphoreType.DMA,
        ],
    ),
    out_shape=jax.ShapeDtypeStruct((CAP, D), jnp.float32),
)

@functools.cache
def _sc_kernels():
    mesh = plsc.VectorSubcoreMesh(core_axis_name="c", subcore_axis_name="s")
    scratch = (
        [pltpu.VMEM((NCH, CH), jnp.int32)]
        + [pltpu.VMEM((CH, D), jnp.float32) for _ in range(NBUF)]
        + [pltpu.SemaphoreType.DMA for _ in range(2 * NBUF)]
    )

    @functools.partial(
        pl.kernel,
        out_type=jax.ShapeDtypeStruct((CAP, D), jnp.float32),
        mesh=mesh,
        scratch_types=scratch,
    )
    def sc_scatter(h_hbm, p_hbm, xs_hbm, idx_v, *bufs_sems):
        bufs = bufs_sems[:NBUF]
        lsems = bufs_sems[NBUF : NBUF + NBUF]
        ssems = bufs_sems[NBUF + NBUF :]
        wid = lax.axis_index("s") * 2 + lax.axis_index("c")
        base = wid * TOK_PER_W
        pltpu.sync_copy(p_hbm.at[pl.ds(wid * NCH, NCH)], idx_v)
        for j in range(DEPTH):
            pltpu.async_copy(
                h_hbm.at[pl.ds(base + j * CH, CH)], bufs[j], lsems[j]
            )
        for c in range(NCH):
            b = c % NBUF
            pltpu.make_async_copy(
                h_hbm.at[pl.ds(base, CH)], bufs[b], lsems[b]
            ).wait()
            pltpu.async_copy(bufs[b], xs_hbm.at[idx_v.at[c]], ssems[b])
            nxt = c + DEPTH
            if nxt < NCH:
                bb = nxt % NBUF
                if nxt >= NBUF:
                    pltpu.make_async_copy(
                        bufs[bb], xs_hbm.at[idx_v.at[c]], ssems[bb]
                    ).wait()
                pltpu.async_copy(
                    h_hbm.at[pl.ds(base + nxt * CH, CH)], bufs[bb], lsems[bb]
                )
        for c in range(NCH - NBUF, NCH):
            b = c % NBUF
            pltpu.make_async_copy(
                bufs[b], xs_hbm.at[idx_v.at[c]], ssems[b]
            ).wait()

    @functools.partial(
        pl.kernel,
        out_type=jax.ShapeDtypeStruct((T, D), jnp.float32),
        mesh=mesh,
        scratch_types=scratch,
    )
    def sc_gather(y_hbm, p_hbm, out_hbm, idx_v, *bufs_sems):
        bufs = bufs_sems[:NBUF]
        gsems = bufs_sems[NBUF : NBUF + NBUF]
        wsems = bufs_sems[NBUF + NBUF :]
        wid = lax.axis_index("s") * 2 + lax.axis_index("c")
        base = wid * TOK_PER_W
        pltpu.sync_copy(p_hbm.at[pl.ds(wid * NCH, NCH)], idx_v)
        for j in range(DEPTH):
            pltpu.async_copy(y_hbm.at[idx_v.at[j]], bufs[j], gsems[j])
        for c in range(NCH):
            b = c % NBUF
            pltpu.make_async_copy(
                y_hbm.at[idx_v.at[c]], bufs[b], gsems[b]
            ).wait()
            pltpu.async_copy(
                bufs[b], out_hbm.at[pl.ds(base + c * CH, CH)], wsems[b]
            )
            nxt = c + DEPTH
            if nxt < NCH:
                bb = nxt % NBUF
                if nxt >= NBUF:
                    pltpu.make_async_copy(
                        bufs[bb], out_hbm.at[pl.ds(base, CH)], wsems[bb]
                    ).wait()
                pltpu.async_copy(y_hbm.at[idx_v.at[nxt]], bufs[bb], gsems[bb])
        for c in range(NCH - NBUF, NCH):
            b = c % NBUF
            pltpu.make_async_copy(
                bufs[b], out_hbm.at[pl.ds(base, CH)], wsems[b]
            ).wait()

    return sc_scatter, sc_gather


@jax.jit
def kernel(hidden_states, type_ids, W, b):
    B, S, _ = hidden_states.shape
    h2d = hidden_states.reshape(T, D)
    tid = type_ids.reshape(TROWS, TLANES).astype(jnp.int32)
    p2d, aux = _routing(tid)
    p_chunks = p2d.reshape(T // CH, CH)
    sc_scatter, sc_gather = _sc_kernels()
    x_sorted = sc_scatter(h2d, p_chunks)
    y_sorted = _grouped_mm(aux, x_sorted, W, b.reshape(E, 1, D))
    out = sc_gather(y_sorted, p_chunks)
    return out.reshape(B, S, D)


# speedup vs baseline: 1.0598x; 1.0037x over previous
import functools

import jax
import jax.numpy as jnp
from jax import lax
from jax.experimental import pallas as pl
from jax.experimental.pallas import tpu as pltpu
from jax.experimental.pallas import tpu_sc as plsc

E = 4
D = 2048
R = 256
T = 4096
MAXB = T // R + E - 1
CAP = MAXB * R

TROWS = 32
TLANES = 128

NW = 32
TOK_PER_W = T // NW
CH = 8
NCH = TOK_PER_W // CH
NBUF = 6
DEPTH = 3


def _routing_body(tid_ref, p_ref, gid_ref):
    tid = tid_ref[...]
    rk = lax.broadcasted_iota(jnp.int32, (TLANES, TLANES), 0)
    ck = lax.broadcasted_iota(jnp.int32, (TLANES, TLANES), 1)
    upper_incl = (rk <= ck).astype(jnp.float32)
    rr = lax.broadcasted_iota(jnp.int32, (TROWS, TROWS), 0)
    cr = lax.broadcasted_iota(jnp.int32, (TROWS, TROWS), 1)
    lower_strict = (cr < rr).astype(jnp.float32)

    ranks = []
    counts = []
    for g in range(E):
        m = (tid == g).astype(jnp.float32)
        lane_cum = jnp.dot(m, upper_incl, preferred_element_type=jnp.float32,
                           precision=lax.Precision.HIGHEST)
        row_tot = jnp.sum(m, axis=1, keepdims=True)
        row_excl = jnp.dot(lower_strict, row_tot,
                           preferred_element_type=jnp.float32,
                           precision=lax.Precision.HIGHEST)
        ranks.append(row_excl + lane_cum - 1.0)
        counts.append(jnp.sum(m))

    p = jnp.zeros((TROWS, TLANES), jnp.float32)
    base = jnp.float32(0.0)
    nblk_cum = []
    acc = jnp.float32(0.0)
    for g in range(E):
        p = jnp.where(tid == g, base + ranks[g], p)
        nblk = jnp.ceil(counts[g] / R)
        base = base + nblk * R
        acc = acc + nblk
        nblk_cum.append(acc)
    p_ref[...] = p.astype(jnp.int32)

    ii = lax.broadcasted_iota(jnp.int32, (1, TLANES), 1).astype(jnp.float32)
    gid = jnp.zeros((1, TLANES), jnp.float32)
    for g in range(E):
        gid = gid + (ii >= nblk_cum[g]).astype(jnp.float32)
    gid = jnp.minimum(gid, float(E - 1))
    nb = nblk_cum[E - 1]

    prev = jnp.concatenate([gid[:, :1], gid[:, :-1]], axis=1)
    valid = ii < nb
    chg = jnp.where((gid != prev) | (ii == 0.0), 1.0, 0.0) * valid
    cs = jnp.dot(chg, upper_incl, preferred_element_type=jnp.float32,
                 precision=lax.Precision.HIGHEST)
    parity = cs - 1.0 - 2.0 * jnp.floor((cs - 1.0) / 2.0)
    np3 = jnp.float32(3.0)
    np2 = jnp.where(counts[3] > 0, 3.0, 2.0)
    np1 = jnp.where(counts[2] > 0, 2.0, jnp.where(counts[3] > 0, 3.0, 1.0))
    np0 = jnp.where(
        counts[1] > 0, 1.0,
        jnp.where(counts[2] > 0, 2.0, jnp.where(counts[3] > 0, 3.0, 0.0)),
    )
    nps = [np0, np1, np2, np3]
    nxt = jnp.zeros((1, TLANES), jnp.float32)
    for g in range(E):
        nxt = jnp.where(gid == g, nps[g], nxt)

    row0 = jnp.where(ii == float(MAXB), nb, gid)
    aux = jnp.concatenate([row0, parity, chg, nxt], axis=0)
    gid_ref[...] = aux.astype(jnp.int32)


_routing = pl.pallas_call(
    _routing_body,
    out_shape=(
        jax.ShapeDtypeStruct((TROWS, TLANES), jnp.int32),
        jax.ShapeDtypeStruct((4, TLANES), jnp.int32),
    ),
)


NQ = 4
KQ = D // NQ


def _mm_body(aux_ref, x_ref, w_hbm, b_ref, y_ref, w_vmem, sem0, sem1, *qsems):
    i = pl.program_id(0)
    nb = aux_ref[0, MAXB]
    gid = aux_ref[0, i]
    par = aux_ref[1, i]
    start = aux_ref[2, i]
    nxt = aux_ref[3, i]
    real = i < nb
    sems = (sem0, sem1)

    def copy_w(g_idx, slot):
        pltpu.make_async_copy(w_hbm.at[g_idx], w_vmem.at[slot], sems[slot]).start()

    @pl.when(real & (i == 0))
    def _():
        for q in range(NQ):
            pltpu.make_async_copy(
                w_hbm.at[gid, pl.ds(q * KQ, KQ)],
                w_vmem.at[0, pl.ds(q * KQ, KQ)],
                qsems[q],
            ).start()

        @pl.when(nxt != gid)
        def _():
            copy_w(nxt, 1)

        x = x_ref[...]
        acc = jnp.broadcast_to(b_ref[0], (R, D))
        for q in range(NQ):
            pltpu.make_async_copy(
                w_hbm.at[0, pl.ds(0, KQ)],
                w_vmem.at[0, pl.ds(0, KQ)],
                qsems[q],
            ).wait()
            acc = acc + jnp.dot(
                x[:, q * KQ : (q + 1) * KQ],
                w_vmem[0, q * KQ : (q + 1) * KQ, :],
                preferred_element_type=jnp.float32,
            )
        y_ref[...] = acc

    for s in (0, 1):
        @pl.when(real & (start == 1) & (par == s) & (i > 0))
        def _(s=s):
            pltpu.make_async_copy(w_hbm.at[0], w_vmem.at[s], sems[s]).wait()

            @pl.when(nxt != gid)
            def _():
                copy_w(nxt, 1 - s)

    for s in (0, 1):
        @pl.when(real & (par == s) & (i > 0))
        def _(s=s):
            y_ref[...] = (
                jnp.dot(x_ref[...], w_vmem[s],
                        preferred_element_type=jnp.float32)
                + b_ref[0]
            )


_grouped_mm = pl.pallas_call(
    _mm_body,
    grid_spec=pltpu.PrefetchScalarGridSpec(
        num_scalar_prefetch=1,
        grid=(MAXB,),
        in_specs=[
            pl.BlockSpec((R, D), lambda i, aux: (i, 0)),
            pl.BlockSpec(memory_space=pl.ANY),
            pl.BlockSpec((1, 1, D), lambda i, aux: (aux[0, i], 0, 0)),
        ],
        out_specs=pl.BlockSpec((R, D), lambda i, aux: (i, 0)),
        scratch_shapes=[pltpu.VMEM((2, D, D), jnp.float32)]
        + [pltpu.SemaphoreType.DMA] * (2 + NQ),
    ),
    out_shape=jax.ShapeDtypeStruct((CAP, D), jnp.float32),
)

@functools.cache
def _sc_kernels():
    mesh = plsc.VectorSubcoreMesh(core_axis_name="c", subcore_axis_name="s")
    scratch = (
        [pltpu.VMEM((NCH, CH), jnp.int32)]
        + [pltpu.VMEM((CH, D), jnp.float32) for _ in range(NBUF)]
        + [pltpu.SemaphoreType.DMA for _ in range(2 * NBUF)]
    )

    @functools.partial(
        pl.kernel,
        out_type=jax.ShapeDtypeStruct((CAP, D), jnp.float32),
        mesh=mesh,
        scratch_types=scratch,
    )
    def sc_scatter(h_hbm, p_hbm, xs_hbm, idx_v, *bufs_sems):
        bufs = bufs_sems[:NBUF]
        lsems = bufs_sems[NBUF : NBUF + NBUF]
        ssems = bufs_sems[NBUF + NBUF :]
        wid = lax.axis_index("s") * 2 + lax.axis_index("c")
        base = wid * TOK_PER_W
        pltpu.sync_copy(p_hbm.at[pl.ds(wid * NCH, NCH)], idx_v)
        for j in range(DEPTH):
            pltpu.async_copy(
                h_hbm.at[pl.ds(base + j * CH, CH)], bufs[j], lsems[j]
            )
        for c in range(NCH):
            b = c % NBUF
            pltpu.make_async_copy(
                h_hbm.at[pl.ds(base, CH)], bufs[b], lsems[b]
            ).wait()
            pltpu.async_copy(bufs[b], xs_hbm.at[idx_v.at[c]], ssems[b])
            nxt = c + DEPTH
            if nxt < NCH:
                bb = nxt % NBUF
                if nxt >= NBUF:
                    pltpu.make_async_copy(
                        bufs[bb], xs_hbm.at[idx_v.at[c]], ssems[bb]
                    ).wait()
                pltpu.async_copy(
                    h_hbm.at[pl.ds(base + nxt * CH, CH)], bufs[bb], lsems[bb]
                )
        for c in range(NCH - NBUF, NCH):
            b = c % NBUF
            pltpu.make_async_copy(
                bufs[b], xs_hbm.at[idx_v.at[c]], ssems[b]
            ).wait()

    @functools.partial(
        pl.kernel,
        out_type=jax.ShapeDtypeStruct((T, D), jnp.float32),
        mesh=mesh,
        scratch_types=scratch,
    )
    def sc_gather(y_hbm, p_hbm, out_hbm, idx_v, *bufs_sems):
        bufs = bufs_sems[:NBUF]
        gsems = bufs_sems[NBUF : NBUF + NBUF]
        wsems = bufs_sems[NBUF + NBUF :]
        wid = lax.axis_index("s") * 2 + lax.axis_index("c")
        base = wid * TOK_PER_W
        pltpu.sync_copy(p_hbm.at[pl.ds(wid * NCH, NCH)], idx_v)
        for j in range(DEPTH):
            pltpu.async_copy(y_hbm.at[idx_v.at[j]], bufs[j], gsems[j])
        for c in range(NCH):
            b = c % NBUF
            pltpu.make_async_copy(
                y_hbm.at[idx_v.at[c]], bufs[b], gsems[b]
            ).wait()
            pltpu.async_copy(
                bufs[b], out_hbm.at[pl.ds(base + c * CH, CH)], wsems[b]
            )
            nxt = c + DEPTH
            if nxt < NCH:
                bb = nxt % NBUF
                if nxt >= NBUF:
                    pltpu.make_async_copy(
                        bufs[bb], out_hbm.at[pl.ds(base, CH)], wsems[bb]
                    ).wait()
                pltpu.async_copy(y_hbm.at[idx_v.at[nxt]], bufs[bb], gsems[bb])
        for c in range(NCH - NBUF, NCH):
            b = c % NBUF
            pltpu.make_async_copy(
                bufs[b], out_hbm.at[pl.ds(base, CH)], wsems[b]
            ).wait()

    return sc_scatter, sc_gather


@jax.jit
def kernel(hidden_states, type_ids, W, b):
    B, S, _ = hidden_states.shape
    h2d = hidden_states.reshape(T, D)
    tid = type_ids.reshape(TROWS, TLANES).astype(jnp.int32)
    p2d, aux = _routing(tid)
    p_chunks = p2d.reshape(T // CH, CH)
    sc_scatter, sc_gather = _sc_kernels()
    x_sorted = sc_scatter(h2d, p_chunks)
    y_sorted = _grouped_mm(aux, x_sorted, W, b.reshape(E, 1, D))
    out = sc_gather(y_sorted, p_chunks)
    return out.reshape(B, S, D)
